# Initial kernel scaffold; baseline (speedup 1.0000x reference)
import jax
import jax.numpy as jnp
from jax.experimental import pallas as pl


def kernel(feature2d, depth_3d, w_ds1, b1_w_in, b1_w133, b1_b133, b1_w331, b1_b331, b1_w313, b1_b313, b1_w_out, w_ds2, b2_w_in, b2_w133, b2_b133, b2_w331, b2_b331, b2_w313, b2_b313, b2_w_out):
    raise NotImplementedError("write your pallas kernel here")


# speedup vs baseline: 1.6968x; 1.6968x over previous
import functools

import jax
import jax.numpy as jnp
from jax import lax
from jax.experimental import pallas as pl
from jax.experimental.pallas import tpu as pltpu

D, H, W = 240, 144, 240
V = D * H * W
DP = 244
HK, WK = 73, 121
VP = DP * 2 * 2 * HK * WK

T = 2
Z = 2 * T + 3
Z2 = 2 * T + 1


def _shift_k_m1(x):
    return jnp.concatenate([jnp.zeros_like(x[..., :1]), x[..., :-1]], axis=-1)


def _shift_k_p1(x):
    return jnp.concatenate([x[..., 1:], jnp.zeros_like(x[..., :1])], axis=-1)


def _shift_a_m1(x):
    return jnp.concatenate([jnp.zeros_like(x[..., :1, :]), x[..., :-1, :]], axis=-2)


def _shift_a_p1(x):
    return jnp.concatenate([x[..., 1:, :], jnp.zeros_like(x[..., :1, :])], axis=-2)


def _fused_kernel(w_ref, p_hbm, out_ref, slab, ybuf, sem):
    i = pl.program_id(0)
    cp = pltpu.make_async_copy(p_hbm.at[:, pl.ds(i * 2 * T, Z)], slab, sem)
    cp.start()
    cp.wait()

    s4 = [[slab[:, :, hb, 0], slab[:, :, hb, 1]] for hb in range(2)]

    wsw = [[None, None], [None, None]]
    for hb in range(2):
        e, o = s4[hb][0], s4[hb][1]
        wsw[hb][0] = e + o + _shift_k_m1(o)
        wsw[hb][1] = o + e + _shift_k_p1(e)
    wsh = [[None, None], [None, None]]
    for wb in range(2):
        e, o = wsw[0][wb], wsw[1][wb]
        wsh[0][wb] = e + o + _shift_a_m1(o)
        wsh[1][wb] = o + e + _shift_a_p1(e)
    k_iota = lax.broadcasted_iota(jnp.int32, (HK, WK), 1)
    a_iota = lax.broadcasted_iota(jnp.int32, (HK, WK), 0)
    wmask = [(k_iota >= 1).astype(jnp.float32), (k_iota <= WK - 2).astype(jnp.float32)]
    hmask = [(a_iota >= 1).astype(jnp.float32), (a_iota <= HK - 2).astype(jnp.float32)]
    dreal = 2 * i * T + jnp.arange(Z2, dtype=jnp.int32) - 1
    dmask = ((dreal >= 0) & (dreal < D)).astype(jnp.float32)[None, :, None, None]

    for hb in range(2):
        for wb in range(2):
            x = wsh[hb][wb]
            ws3 = x[:, 0:Z2] + x[:, 1:Z2 + 1] + x[:, 2:Z2 + 2]
            s_in = s4[hb][wb][:, 1:Z2 + 1]
            y = s_in + ws3 * (1.0 / 27.0) * (s_in == 0.0).astype(jnp.float32)
            y = y * (wmask[wb] * hmask[hb])[None, None] * dmask
            ybuf[hb, wb] = y

    out_ref[0:4] = jnp.zeros((4, T, 72, 120), jnp.float32)

    def conv_step(t, carry):
        dj = t // 3
        kd = t - 3 * dj
        dz = 2 * dj + kd
        for kh in range(3):
            hb, a0 = (kh % 2, kh // 2)
            for kw in range(3):
                wb, k0 = (kw % 2, kw // 2)
                ysl = ybuf[hb, wb, :, dz, a0:a0 + 72, k0:k0 + 120]
                for o in range(4):
                    acc = out_ref[o, dj]
                    for c in range(12):
                        acc = acc + ysl[c] * w_ref[o, c, kd, kh, kw]
                    out_ref[o, dj] = acc
        return carry

    lax.fori_loop(0, 3 * T, conv_step, 0, unroll=False)
    out_ref[0:4] = jnp.maximum(out_ref[0:4], 0.0)

    for dj in range(T):
        m = None
        for dz in (2 * dj + 1, 2 * dj + 2):
            for hb, a0 in ((1, 0), (0, 1)):
                for wb, k0 in ((1, 0), (0, 1)):
                    ysl = ybuf[hb, wb, :, dz, a0:a0 + 72, k0:k0 + 120]
                    m = ysl if m is None else jnp.maximum(m, ysl)
        out_ref[4:16, dj] = jnp.maximum(m, 0.0)


@functools.partial(jax.jit, static_argnums=())
def _fused_front(pbuf, w_ds1):
    grid = (D // 2 // T,)
    out = pl.pallas_call(
        _fused_kernel,
        grid=grid,
        in_specs=[
            pl.BlockSpec(memory_space=pltpu.SMEM),
            pl.BlockSpec(memory_space=pltpu.HBM),
        ],
        out_specs=pl.BlockSpec((16, T, 72, 120), lambda i: (0, i, 0, 0)),
        out_shape=jax.ShapeDtypeStruct((16, 120, 72, 120), jnp.float32),
        scratch_shapes=[
            pltpu.VMEM((12, Z, 2, 2, HK, WK), jnp.float32),
            pltpu.VMEM((2, 2, 12, Z2, HK, WK), jnp.float32),
            pltpu.SemaphoreType.DMA,
        ],
        compiler_params=pltpu.CompilerParams(
            dimension_semantics=("arbitrary",),
        ),
    )(w_ds1, pbuf)
    return out


def _conv3d(x, w, padding, stride=(1, 1, 1), b=None):
    y = lax.conv_general_dilated(x, w, window_strides=stride, padding=padding,
                                 dimension_numbers=('NCDHW', 'OIDHW', 'NCDHW'))
    if b is not None:
        y = y + b[None, :, None, None, None]
    return y


def _downsample(x, w):
    c = _conv3d(x, w, [(1, 1)] * 3, stride=(2, 2, 2))
    p = lax.reduce_window(x, -jnp.inf, lax.max, (1, 1, 2, 2, 2), (1, 1, 2, 2, 2), 'VALID')
    return jax.nn.relu(jnp.concatenate([c, p], axis=1))


def _bottleneck(x, w_in, w133, b133, w331, b331, w313, b313, w_out):
    y0 = jax.nn.relu(_conv3d(x, w_in, [(0, 0)] * 3))
    y1 = jax.nn.relu(_conv3d(y0, w133, [(0, 0), (1, 1), (1, 1)], b=b133))
    y2 = jax.nn.relu(_conv3d(y1, w331, [(1, 1), (1, 1), (0, 0)], b=b331) + y1)
    y3 = jax.nn.relu(_conv3d(y2, w313, [(1, 1), (0, 0), (1, 1)], b=b313) + y2 + y1)
    return jax.nn.relu(_conv3d(y3, w_out, [(0, 0)] * 3) + x)


def kernel(feature2d, depth_3d, w_ds1, b1_w_in, b1_w133, b1_b133, b1_w331, b1_b331,
           b1_w313, b1_b313, b1_w_out, w_ds2, b2_w_in, b2_w133, b2_b133, b2_w331,
           b2_b331, b2_w313, b2_b313, b2_w_out):
    b, c, h2, w2 = feature2d.shape
    feat = feature2d.reshape(c, h2 * w2)
    depth = depth_3d.reshape(-1).astype(jnp.int32)

    dd = depth // (H * W)
    r = depth % (H * W)
    hh = r // W
    ww = r % W
    hp = hh + 1
    wp = ww + 1
    pidx = ((((dd + 2) * 2 + (hp % 2)) * 2 + (wp % 2)) * HK + hp // 2) * WK + wp // 2
    pidx = jnp.where(depth > 0, pidx, VP)

    pbuf = jnp.zeros((c, VP), jnp.float32).at[:, pidx].set(feat, mode='drop')
    pbuf = pbuf.reshape(c, DP, 2, 2, HK, WK)

    x = _fused_front(pbuf, w_ds1)[None]
    x = _bottleneck(x, b1_w_in, b1_w133, b1_b133, b1_w331, b1_b331, b1_w313, b1_b313, b1_w_out)
    x = _downsample(x, w_ds2)
    x = _bottleneck(x, b2_w_in, b2_w133, b2_b133, b2_w331, b2_b331, b2_w313, b2_b313, b2_w_out)
    return x


# speedup vs baseline: 1.7509x; 1.0318x over previous
import functools

import jax
import jax.numpy as jnp
from jax import lax
from jax.experimental import pallas as pl
from jax.experimental.pallas import tpu as pltpu

D, H, W = 240, 144, 240
V = D * H * W
DP = 244
HK, WK = 73, 121
VP = DP * 2 * 2 * HK * WK

T = 2
Z = 2 * T + 3
Z2 = 2 * T + 1


def _shift_k_m1(x):
    return jnp.concatenate([jnp.zeros_like(x[..., :1]), x[..., :-1]], axis=-1)


def _shift_k_p1(x):
    return jnp.concatenate([x[..., 1:], jnp.zeros_like(x[..., :1])], axis=-1)


def _shift_a_m1(x):
    return jnp.concatenate([jnp.zeros_like(x[..., :1, :]), x[..., :-1, :]], axis=-2)


def _shift_a_p1(x):
    return jnp.concatenate([x[..., 1:, :], jnp.zeros_like(x[..., :1, :])], axis=-2)


def _fused_kernel(w_ref, p_hbm, out_ref, slab, ybuf, sem):
    i = pl.program_id(0)
    cp = pltpu.make_async_copy(p_hbm.at[:, pl.ds(i * 2 * T, Z)], slab, sem)
    cp.start()
    cp.wait()

    s4 = [[slab[:, :, hb, 0].astype(jnp.float32),
           slab[:, :, hb, 1].astype(jnp.float32)] for hb in range(2)]

    wsw = [[None, None], [None, None]]
    for hb in range(2):
        e, o = s4[hb][0], s4[hb][1]
        wsw[hb][0] = e + o + _shift_k_m1(o)
        wsw[hb][1] = o + e + _shift_k_p1(e)
    wsh = [[None, None], [None, None]]
    for wb in range(2):
        e, o = wsw[0][wb], wsw[1][wb]
        wsh[0][wb] = e + o + _shift_a_m1(o)
        wsh[1][wb] = o + e + _shift_a_p1(e)
    k_iota = lax.broadcasted_iota(jnp.int32, (HK, WK), 1)
    a_iota = lax.broadcasted_iota(jnp.int32, (HK, WK), 0)
    wmask = [(k_iota >= 1).astype(jnp.float32), (k_iota <= WK - 2).astype(jnp.float32)]
    hmask = [(a_iota >= 1).astype(jnp.float32), (a_iota <= HK - 2).astype(jnp.float32)]
    dreal = 2 * i * T + jnp.arange(Z2, dtype=jnp.int32) - 1
    dmask = ((dreal >= 0) & (dreal < D)).astype(jnp.float32)[None, :, None, None]

    for hb in range(2):
        for wb in range(2):
            x = wsh[hb][wb]
            ws3 = x[:, 0:Z2] + x[:, 1:Z2 + 1] + x[:, 2:Z2 + 2]
            s_in = s4[hb][wb][:, 1:Z2 + 1]
            y = s_in + ws3 * (1.0 / 27.0) * (s_in == 0.0).astype(jnp.float32)
            y = y * (wmask[wb] * hmask[hb])[None, None] * dmask
            ybuf[hb, wb] = y

    out_ref[0:4] = jnp.zeros((4, T, 72, 120), jnp.float32)

    def conv_step(t, carry):
        dj = t // 3
        kd = t - 3 * dj
        dz = 2 * dj + kd
        for kh in range(3):
            hb, a0 = (kh % 2, kh // 2)
            for kw in range(3):
                wb, k0 = (kw % 2, kw // 2)
                ysl = ybuf[hb, wb, :, dz, a0:a0 + 72, k0:k0 + 120]
                for o in range(4):
                    acc = out_ref[o, dj]
                    for c in range(12):
                        acc = acc + ysl[c] * w_ref[o, c, kd, kh, kw]
                    out_ref[o, dj] = acc
        return carry

    lax.fori_loop(0, 3 * T, conv_step, 0, unroll=False)
    out_ref[0:4] = jnp.maximum(out_ref[0:4], 0.0)

    for dj in range(T):
        m = None
        for dz in (2 * dj + 1, 2 * dj + 2):
            for hb, a0 in ((1, 0), (0, 1)):
                for wb, k0 in ((1, 0), (0, 1)):
                    ysl = ybuf[hb, wb, :, dz, a0:a0 + 72, k0:k0 + 120]
                    m = ysl if m is None else jnp.maximum(m, ysl)
        out_ref[4:16, dj] = jnp.maximum(m, 0.0)


@functools.partial(jax.jit, static_argnums=())
def _fused_front(pbuf, w_ds1):
    grid = (D // 2 // T,)
    out = pl.pallas_call(
        _fused_kernel,
        grid=grid,
        in_specs=[
            pl.BlockSpec(memory_space=pltpu.SMEM),
            pl.BlockSpec(memory_space=pltpu.HBM),
        ],
        out_specs=pl.BlockSpec((16, T, 72, 120), lambda i: (0, i, 0, 0)),
        out_shape=jax.ShapeDtypeStruct((16, 120, 72, 120), jnp.float32),
        scratch_shapes=[
            pltpu.VMEM((12, Z, 2, 2, HK, WK), jnp.bfloat16),
            pltpu.VMEM((2, 2, 12, Z2, HK, WK), jnp.float32),
            pltpu.SemaphoreType.DMA,
        ],
        compiler_params=pltpu.CompilerParams(
            dimension_semantics=("arbitrary",),
        ),
    )(w_ds1, pbuf)
    return out


def _conv3d(x, w, padding, stride=(1, 1, 1), b=None):
    y = lax.conv_general_dilated(x, w, window_strides=stride, padding=padding,
                                 dimension_numbers=('NCDHW', 'OIDHW', 'NCDHW'))
    if b is not None:
        y = y + b[None, :, None, None, None]
    return y


def _downsample(x, w):
    c = _conv3d(x, w, [(1, 1)] * 3, stride=(2, 2, 2))
    p = lax.reduce_window(x, -jnp.inf, lax.max, (1, 1, 2, 2, 2), (1, 1, 2, 2, 2), 'VALID')
    return jax.nn.relu(jnp.concatenate([c, p], axis=1))


def _bottleneck(x, w_in, w133, b133, w331, b331, w313, b313, w_out):
    y0 = jax.nn.relu(_conv3d(x, w_in, [(0, 0)] * 3))
    y1 = jax.nn.relu(_conv3d(y0, w133, [(0, 0), (1, 1), (1, 1)], b=b133))
    y2 = jax.nn.relu(_conv3d(y1, w331, [(1, 1), (1, 1), (0, 0)], b=b331) + y1)
    y3 = jax.nn.relu(_conv3d(y2, w313, [(1, 1), (0, 0), (1, 1)], b=b313) + y2 + y1)
    return jax.nn.relu(_conv3d(y3, w_out, [(0, 0)] * 3) + x)


def kernel(feature2d, depth_3d, w_ds1, b1_w_in, b1_w133, b1_b133, b1_w331, b1_b331,
           b1_w313, b1_b313, b1_w_out, w_ds2, b2_w_in, b2_w133, b2_b133, b2_w331,
           b2_b331, b2_w313, b2_b313, b2_w_out):
    b, c, h2, w2 = feature2d.shape
    feat = feature2d.reshape(c, h2 * w2)
    depth = depth_3d.reshape(-1).astype(jnp.int32)

    dd = depth // (H * W)
    r = depth % (H * W)
    hh = r // W
    ww = r % W
    hp = hh + 1
    wp = ww + 1
    pidx = ((((dd + 2) * 2 + (hp % 2)) * 2 + (wp % 2)) * HK + hp // 2) * WK + wp // 2
    pidx = jnp.where(depth > 0, pidx, VP)

    pbuf = jnp.zeros((c, VP), jnp.bfloat16).at[:, pidx].set(
        feat.astype(jnp.bfloat16), mode='drop')
    pbuf = pbuf.reshape(c, DP, 2, 2, HK, WK)

    x = _fused_front(pbuf, w_ds1)[None]
    x = _bottleneck(x, b1_w_in, b1_w133, b1_b133, b1_w331, b1_b331, b1_w313, b1_b313, b1_w_out)
    x = _downsample(x, w_ds2)
    x = _bottleneck(x, b2_w_in, b2_w133, b2_b133, b2_w331, b2_b331, b2_w313, b2_b313, b2_w_out)
    return x
